# initial kernel scaffold (unmeasured)
import jax
import jax.numpy as jnp
from jax import lax
from jax.experimental import pallas as pl
from jax.experimental.pallas import tpu as pltpu

N_DEV = 8
M_HALF = 512
K = 8192


def kernel(x, w_mat):
    m_per, k = x.shape
    _, n_per = w_mat.shape
    n_hops = N_DEV - 1

    def body(x_ref, w_ref, out_ref, buf_a, buf_b, copy_sems,
             send_a, recv_a, send_b, recv_b):
        my = lax.axis_index("i")
        right = lax.rem(my + 1, N_DEV)
        left = lax.rem(my + N_DEV - 1, N_DEV)

        cp_a = pltpu.make_async_copy(
            x_ref.at[pl.ds(0, M_HALF), :], buf_a.at[0], copy_sems.at[0])
        cp_b = pltpu.make_async_copy(
            x_ref.at[pl.ds(M_HALF, M_HALF), :], buf_b.at[0], copy_sems.at[1])
        cp_a.start()
        cp_b.start()
        cp_a.wait()
        cp_b.wait()

        barrier_sem = pltpu.get_barrier_semaphore()
        pl.semaphore_signal(barrier_sem, inc=1, device_id=(left,),
                            device_id_type=pl.DeviceIdType.MESH)
        pl.semaphore_signal(barrier_sem, inc=1, device_id=(right,),
                            device_id_type=pl.DeviceIdType.MESH)
        pl.semaphore_wait(barrier_sem, 2)

        def make_hop(h):
            s, r = h % 2, (h + 1) % 2
            rdma_a = pltpu.make_async_remote_copy(
                src_ref=buf_a.at[s], dst_ref=buf_a.at[r],
                send_sem=send_a.at[h], recv_sem=recv_a.at[h],
                device_id=(right,), device_id_type=pl.DeviceIdType.MESH)
            rdma_b = pltpu.make_async_remote_copy(
                src_ref=buf_b.at[s], dst_ref=buf_b.at[r],
                send_sem=send_b.at[h], recv_sem=recv_b.at[h],
                device_id=(left,), device_id_type=pl.DeviceIdType.MESH)
            return rdma_a, rdma_b

        def gemm(src_buf, slot, origin, half):
            row0 = origin * m_per + half * M_HALF
            acc = jnp.dot(src_buf[slot], w_ref[:, :],
                          preferred_element_type=jnp.float32)
            out_ref[pl.ds(row0, M_HALF), :] = acc

        rdma_a, rdma_b = make_hop(0)
        rdma_a.start()
        rdma_b.start()
        gemm(buf_a, 0, my, 0)
        gemm(buf_b, 0, my, 1)

        for h in range(n_hops):
            rdma_a.wait()
            rdma_b.wait()
            recv_slot = (h + 1) % 2
            if h + 1 < n_hops:
                rdma_a, rdma_b = make_hop(h + 1)
                rdma_a.start()
                rdma_b.start()
            origin_a = lax.rem(my + N_DEV - (h + 1), N_DEV)
            origin_b = lax.rem(my + (h + 1), N_DEV)
            gemm(buf_a, recv_slot, origin_a, 0)
            gemm(buf_b, recv_slot, origin_b, 1)

    return pl.pallas_call(
        body,
        out_shape=jax.ShapeDtypeStruct((N_DEV * m_per, n_per), jnp.float32),
        in_specs=[
            pl.BlockSpec(memory_space=pltpu.ANY),
            pl.BlockSpec(memory_space=pltpu.VMEM),
        ],
        out_specs=pl.BlockSpec(memory_space=pltpu.VMEM),
        scratch_shapes=[
            pltpu.VMEM((2, M_HALF, k), jnp.float32),
            pltpu.VMEM((2, M_HALF, k), jnp.float32),
            pltpu.SemaphoreType.DMA((2,)),
            pltpu.SemaphoreType.DMA((n_hops,)),
            pltpu.SemaphoreType.DMA((n_hops,)),
            pltpu.SemaphoreType.DMA((n_hops,)),
            pltpu.SemaphoreType.DMA((n_hops,)),
        ],
        compiler_params=pltpu.CompilerParams(
            collective_id=0,
            vmem_limit_bytes=128 * 1024 * 1024,
        ),
    )(x, w_mat)


# baseline (device time: 1325732 ns/iter reference)
import jax
import jax.numpy as jnp
from jax import lax
from jax.experimental import pallas as pl
from jax.experimental.pallas import tpu as pltpu

N_DEV = 8
M_HALF = 512
N_HOPS = N_DEV - 1


def kernel(x, w_mat):
    m_per, k = x.shape
    _, n_per = w_mat.shape

    def body(x_ref, w_ref, out_ref, hbm_a, hbm_b, stage, acc,
             send_a, recv_a, send_b, recv_b, loc_sems):
        my = lax.axis_index("i")
        right = lax.rem(my + 1, N_DEV)
        left = lax.rem(my + N_DEV - 1, N_DEV)

        barrier_sem = pltpu.get_barrier_semaphore()
        pl.semaphore_signal(barrier_sem, inc=1, device_id=(left,),
                            device_id_type=pl.DeviceIdType.MESH)
        pl.semaphore_signal(barrier_sem, inc=1, device_id=(right,),
                            device_id_type=pl.DeviceIdType.MESH)
        pl.semaphore_wait(barrier_sem, 2)

        def rdma(src, dst, ssem, rsem, target):
            return pltpu.make_async_remote_copy(
                src_ref=src, dst_ref=dst, send_sem=ssem, recv_sem=rsem,
                device_id=(target,), device_id_type=pl.DeviceIdType.MESH)

        x_a = x_ref.at[pl.ds(0, M_HALF), :]
        x_b = x_ref.at[pl.ds(M_HALF, M_HALF), :]
        rdma_a = rdma(x_a, hbm_a.at[0], send_a.at[0], recv_a.at[0], right)
        rdma_b = rdma(x_b, hbm_b.at[0], send_b.at[0], recv_b.at[0], left)
        rdma_a.start()
        rdma_b.start()

        def tile(src, slot, origin, half):
            cp_in = pltpu.make_async_copy(src, stage.at[slot],
                                          loc_sems.at[slot])
            cp_in.start()
            cp_in.wait()
            acc[slot] = jnp.dot(stage[slot], w_ref[:, :],
                                preferred_element_type=jnp.float32)
            row0 = origin * m_per + half * M_HALF
            cp_out = pltpu.make_async_copy(
                acc.at[slot], out_ref.at[pl.ds(row0, M_HALF), :],
                loc_sems.at[slot])
            cp_out.start()
            cp_out.wait()

        tile(x_a, 0, my, 0)
        tile(x_b, 1, my, 1)

        sends = [(rdma_a, rdma_b)]
        for h in range(N_HOPS):
            pltpu.make_async_copy(hbm_a.at[h], hbm_a.at[h],
                                  recv_a.at[h]).wait()
            pltpu.make_async_copy(hbm_b.at[h], hbm_b.at[h],
                                  recv_b.at[h]).wait()
            if h + 1 < N_HOPS:
                rdma_a = rdma(hbm_a.at[h], hbm_a.at[h + 1],
                              send_a.at[h + 1], recv_a.at[h + 1], right)
                rdma_b = rdma(hbm_b.at[h], hbm_b.at[h + 1],
                              send_b.at[h + 1], recv_b.at[h + 1], left)
                rdma_a.start()
                rdma_b.start()
                sends.append((rdma_a, rdma_b))
            origin_a = lax.rem(my + N_DEV - (h + 1), N_DEV)
            origin_b = lax.rem(my + (h + 1), N_DEV)
            tile(hbm_a.at[h], 0, origin_a, 0)
            tile(hbm_b.at[h], 1, origin_b, 1)

        for sa, sb in sends:
            sa.wait_send()
            sb.wait_send()

    out, _, _ = pl.pallas_call(
        body,
        out_shape=[
            jax.ShapeDtypeStruct((N_DEV * m_per, n_per), jnp.float32),
            jax.ShapeDtypeStruct((N_HOPS, M_HALF, k), jnp.float32),
            jax.ShapeDtypeStruct((N_HOPS, M_HALF, k), jnp.float32),
        ],
        in_specs=[
            pl.BlockSpec(memory_space=pl.ANY),
            pl.BlockSpec(memory_space=pltpu.VMEM),
        ],
        out_specs=[
            pl.BlockSpec(memory_space=pl.ANY),
            pl.BlockSpec(memory_space=pl.ANY),
            pl.BlockSpec(memory_space=pl.ANY),
        ],
        scratch_shapes=[
            pltpu.VMEM((2, M_HALF, k), jnp.float32),
            pltpu.VMEM((2, M_HALF, n_per), jnp.float32),
            pltpu.SemaphoreType.DMA((N_HOPS,)),
            pltpu.SemaphoreType.DMA((N_HOPS,)),
            pltpu.SemaphoreType.DMA((N_HOPS,)),
            pltpu.SemaphoreType.DMA((N_HOPS,)),
            pltpu.SemaphoreType.DMA((2,)),
        ],
        compiler_params=pltpu.CompilerParams(
            collective_id=0,
            vmem_limit_bytes=60 * 1024 * 1024,
        ),
    )(x, w_mat)
    return out
